# Initial kernel scaffold; baseline (speedup 1.0000x reference)
import functools

import jax
import jax.numpy as jnp
from jax import lax
from jax.experimental import pallas as pl
from jax.experimental.pallas import tpu as pltpu
from jax.experimental.pallas import tpu_sc as plsc

B, S, D = 4, 4096, 1024
NS = 256
K = 8
NB = S // K
NLVL = 10
L = 16
NV = D // L
MAXROWS = 16
NW = 32
SPW = (B * NS) // NW
NEG = float(jnp.finfo(jnp.float32).min)
DH = 256


def _table_body(x_ref, t_ref):
    x = x_ref[0]
    cur = jnp.max(x.reshape(NB, K, DH), axis=1)
    t_ref[0, 0:NB] = cur
    for l in range(1, NLVL):
        h = 1 << (l - 1)
        cur = jnp.maximum(cur, jnp.concatenate([cur[h:], cur[:h]], axis=0))
        t_ref[0, l * NB:(l + 1) * NB] = cur


def _build_table(context):
    return pl.pallas_call(
        _table_body,
        grid=(B, D // DH),
        in_specs=[pl.BlockSpec((1, S, DH), lambda b, d: (b, 0, d))],
        out_specs=pl.BlockSpec((1, NLVL * NB, DH), lambda b, d: (b, 0, d)),
        out_shape=jax.ShapeDtypeStruct((B, NLVL * NB, D), jnp.float32),
    )(context)


def _sc_pool(context, table, starts, ends):
    mesh = plsc.VectorSubcoreMesh(core_axis_name="c", subcore_axis_name="s")

    @functools.partial(
        pl.kernel,
        out_type=jax.ShapeDtypeStruct((B, NS, D), jnp.float32),
        mesh=mesh,
        scratch_types=[
            pltpu.VMEM((SPW,), jnp.int32),
            pltpu.VMEM((SPW,), jnp.int32),
            pltpu.VMEM((MAXROWS, D), jnp.float32),
            pltpu.VMEM((D,), jnp.float32),
            pltpu.SemaphoreType.DMA,
        ],
    )
    def pool(ctx_hbm, tab_hbm, st_hbm, en_hbm, out_hbm,
             st_v, en_v, rows_v, acc_v, sem):
        wid = lax.axis_index("s") * 2 + lax.axis_index("c")
        base = wid * SPW
        b = base // NS
        r0 = base % NS
        pltpu.sync_copy(st_hbm.at[pl.ds(base, SPW)], st_v)
        pltpu.sync_copy(en_hbm.at[pl.ds(base, SPW)], en_v)
        lanes = lax.iota(jnp.int32, L)

        def get(vref, j):
            v = jnp.where(j >= L, vref[pl.ds(L, L)], vref[pl.ds(0, L)])
            return jnp.max(jnp.where(lanes == j % L, v, 0))

        def span_body(j, _):
            s = get(st_v, j)
            e = get(en_v, j)
            s8 = (s + K - 1) // K
            e8 = e // K
            nb = e8 - s8
            has_interior = nb > 0
            lvl = lax.while_loop(lambda l_: (2 << l_) <= nb,
                                 lambda l_: l_ + 1, 0)
            w = 1 << lvl
            n1 = jnp.minimum(e, s8 * K) - s
            lo2 = jnp.maximum(s, e8 * K)
            n2 = e - lo2

            @pl.when(has_interior)
            def _():
                pltpu.async_copy(tab_hbm.at[b, lvl * NB + s8],
                                 rows_v.at[MAXROWS - 2], sem)
                pltpu.async_copy(tab_hbm.at[b, lvl * NB + e8 - w],
                                 rows_v.at[MAXROWS - 1], sem)

            def fire1(i, c):
                pltpu.async_copy(ctx_hbm.at[b, s + i], rows_v.at[i], sem)
                return c
            lax.fori_loop(0, n1, fire1, 0)

            def fire2(i, c):
                pltpu.async_copy(ctx_hbm.at[b, lo2 + i], rows_v.at[n1 + i], sem)
                return c
            lax.fori_loop(0, n2, fire2, 0)

            init = jnp.where(e > s,
                             jnp.full((L,), NEG, jnp.float32),
                             jnp.zeros((L,), jnp.float32))
            for m in range(NV):
                acc_v[pl.ds(m * L, L)] = init

            ntot = n1 + n2 + jnp.where(has_interior, 2, 0)

            def drain(i, c):
                pltpu.make_async_copy(ctx_hbm.at[b, 0], rows_v.at[0], sem).wait()
                return c
            lax.fori_loop(0, ntot, drain, 0)

            def acc_row(i, c):
                for m in range(NV):
                    sl = pl.ds(m * L, L)
                    acc_v[sl] = jnp.maximum(acc_v[sl], rows_v[i, sl])
                return c
            lax.fori_loop(0, n1 + n2, acc_row, 0)

            @pl.when(has_interior)
            def _():
                for m in range(NV):
                    sl = pl.ds(m * L, L)
                    acc_v[sl] = jnp.maximum(
                        jnp.maximum(acc_v[sl], rows_v[MAXROWS - 2, sl]),
                        rows_v[MAXROWS - 1, sl])

            pltpu.sync_copy(acc_v, out_hbm.at[b, r0 + j])
            return 0

        lax.fori_loop(0, SPW, span_body, 0)

    return pool(context, table, starts, ends)


def kernel(context, spans):
    spans = spans.astype(jnp.int32)
    starts = spans[:, :, 0].reshape(B * NS)
    ends = spans[:, :, 1].reshape(B * NS)
    table = _build_table(context)
    pooled = _sc_pool(context, table, starts, ends)
    return jnp.zeros((B, S, D), jnp.float32).at[:, :NS, :].set(pooled)


# speedup vs baseline: 21.0745x; 21.0745x over previous
import dataclasses
import functools

import jax
import jax.numpy as jnp
from jax import lax
from jax.experimental import pallas as pl
from jax.experimental.pallas import tpu as pltpu
from jax.experimental.pallas import tpu_sc as plsc

B, S, D = 4, 4096, 1024
NS = 256
K = 8
NB = S // K
NLVL = 10
L = 16
NV = D // L
MAXROWS = 16
NW = 32
SPW = (B * NS) // NW
NEG = float(jnp.finfo(jnp.float32).min)
DH = 256


def _table_body(x_ref, t_ref):
    x = x_ref[0]
    cur = jnp.max(x.reshape(NB, K, DH), axis=1)
    t_ref[0, 0:NB] = cur
    for l in range(1, NLVL):
        h = 1 << (l - 1)
        cur = jnp.maximum(cur, jnp.concatenate([cur[h:], cur[:h]], axis=0))
        t_ref[0, l * NB:(l + 1) * NB] = cur


def _build_table(context):
    return pl.pallas_call(
        _table_body,
        grid=(B, D // DH),
        in_specs=[pl.BlockSpec((1, S, DH), lambda b, d: (b, 0, d))],
        out_specs=pl.BlockSpec((1, NLVL * NB, DH), lambda b, d: (b, 0, d)),
        out_shape=jax.ShapeDtypeStruct((B, NLVL * NB, D), jnp.float32),
    )(context)


def _sc_pool(context, table, starts, ends):
    mesh = plsc.VectorSubcoreMesh(core_axis_name="c", subcore_axis_name="s")
    cp = pltpu.CompilerParams()
    if "needs_layout_passes" in pltpu.CompilerParams.__dataclass_fields__:
        cp = dataclasses.replace(cp, needs_layout_passes=False)

    @functools.partial(
        pl.kernel,
        out_type=jax.ShapeDtypeStruct((B, NS, D), jnp.float32),
        mesh=mesh,
        compiler_params=cp,
        scratch_types=[
            pltpu.VMEM((SPW,), jnp.int32),
            pltpu.VMEM((SPW,), jnp.int32),
            pltpu.VMEM((MAXROWS, D), jnp.float32),
            pltpu.VMEM((D,), jnp.float32),
            pltpu.SemaphoreType.DMA,
        ],
    )
    def pool(ctx_hbm, tab_hbm, st_hbm, en_hbm, out_hbm,
             st_v, en_v, rows_v, acc_v, sem):
        wid = lax.axis_index("s") * 2 + lax.axis_index("c")
        base = wid * SPW
        b = base // NS
        r0 = base % NS
        pltpu.sync_copy(st_hbm.at[pl.ds(base, SPW)], st_v)
        pltpu.sync_copy(en_hbm.at[pl.ds(base, SPW)], en_v)
        lanes = lax.iota(jnp.int32, L)

        def get(vref, j):
            v = jnp.where(j >= L, vref[pl.ds(L, L)], vref[pl.ds(0, L)])
            return jnp.max(jnp.where(lanes == j % L, v, 0))

        def span_body(j, _):
            s = get(st_v, j)
            e = get(en_v, j)
            s8 = (s + K - 1) // K
            e8 = e // K
            nb = e8 - s8
            has_interior = nb > 0
            lvl = lax.while_loop(lambda l_: (2 << l_) <= nb,
                                 lambda l_: l_ + 1, 0)
            w = 1 << lvl
            n1 = jnp.minimum(e, s8 * K) - s
            lo2 = jnp.maximum(s, e8 * K)
            n2 = e - lo2

            @pl.when(has_interior)
            def _():
                pltpu.async_copy(tab_hbm.at[b, lvl * NB + s8],
                                 rows_v.at[MAXROWS - 2], sem)
                pltpu.async_copy(tab_hbm.at[b, lvl * NB + e8 - w],
                                 rows_v.at[MAXROWS - 1], sem)

            def fire1(i, c):
                pltpu.async_copy(ctx_hbm.at[b, s + i], rows_v.at[i], sem)
                return c
            lax.fori_loop(0, n1, fire1, 0)

            def fire2(i, c):
                pltpu.async_copy(ctx_hbm.at[b, lo2 + i], rows_v.at[n1 + i], sem)
                return c
            lax.fori_loop(0, n2, fire2, 0)

            init = jnp.where(e > s,
                             jnp.full((L,), NEG, jnp.float32),
                             jnp.zeros((L,), jnp.float32))
            for m in range(NV):
                acc_v[pl.ds(m * L, L)] = init

            ntot = n1 + n2 + jnp.where(has_interior, 2, 0)

            def drain(i, c):
                pltpu.make_async_copy(ctx_hbm.at[b, 0], rows_v.at[0], sem).wait()
                return c
            lax.fori_loop(0, ntot, drain, 0)

            def acc_row(i, c):
                for m in range(NV):
                    sl = pl.ds(m * L, L)
                    acc_v[sl] = jnp.maximum(acc_v[sl], rows_v[i, sl])
                return c
            lax.fori_loop(0, n1 + n2, acc_row, 0)

            @pl.when(has_interior)
            def _():
                for m in range(NV):
                    sl = pl.ds(m * L, L)
                    acc_v[sl] = jnp.maximum(
                        jnp.maximum(acc_v[sl], rows_v[MAXROWS - 2, sl]),
                        rows_v[MAXROWS - 1, sl])

            pltpu.sync_copy(acc_v, out_hbm.at[b, r0 + j])
            return 0

        lax.fori_loop(0, SPW, span_body, 0)

    return pool(context, table, starts, ends)


def kernel(context, spans):
    spans = spans.astype(jnp.int32)
    starts = spans[:, :, 0].reshape(B * NS)
    ends = spans[:, :, 1].reshape(B * NS)
    table = _build_table(context)
    pooled = _sc_pool(context, table, starts, ends)
    return jnp.zeros((B, S, D), jnp.float32).at[:, :NS, :].set(pooled)


# speedup vs baseline: 33.8459x; 1.6060x over previous
import dataclasses
import functools

import jax
import jax.numpy as jnp
from jax import lax
from jax.experimental import pallas as pl
from jax.experimental.pallas import tpu as pltpu
from jax.experimental.pallas import tpu_sc as plsc

B, S, D = 4, 4096, 1024
NS = 256
K = 8
NB = S // K
NLVL = 10
L = 16
NV = D // L
MAXROWS = 16
NW = 32
SPW = (B * NS) // NW
NEG = float(jnp.finfo(jnp.float32).min)
DH = 256


def _table_body(x_ref, t_ref):
    x = x_ref[0]
    cur = jnp.max(x.reshape(NB, K, DH), axis=1)
    t_ref[0, 0:NB] = cur
    for l in range(1, NLVL):
        h = 1 << (l - 1)
        cur = jnp.maximum(cur, jnp.concatenate([cur[h:], cur[:h]], axis=0))
        t_ref[0, l * NB:(l + 1) * NB] = cur


def _build_table(context):
    return pl.pallas_call(
        _table_body,
        grid=(B, D // DH),
        in_specs=[pl.BlockSpec((1, S, DH), lambda b, d: (b, 0, d))],
        out_specs=pl.BlockSpec((1, NLVL * NB, DH), lambda b, d: (b, 0, d)),
        out_shape=jax.ShapeDtypeStruct((B, NLVL * NB, D), jnp.float32),
    )(context)


def _sc_pool(context, table, starts, ends):
    mesh = plsc.VectorSubcoreMesh(core_axis_name="c", subcore_axis_name="s")
    cp = pltpu.CompilerParams()
    if "needs_layout_passes" in pltpu.CompilerParams.__dataclass_fields__:
        cp = dataclasses.replace(cp, needs_layout_passes=False)

    @functools.partial(
        pl.kernel,
        out_type=jax.ShapeDtypeStruct((B, NS, D), jnp.float32),
        mesh=mesh,
        compiler_params=cp,
        scratch_types=[
            pltpu.VMEM((SPW,), jnp.int32),
            pltpu.VMEM((SPW,), jnp.int32),
            pltpu.VMEM((2, MAXROWS, D), jnp.float32),
            pltpu.VMEM((D,), jnp.float32),
            pltpu.SemaphoreType.DMA,
            pltpu.SemaphoreType.DMA,
        ],
    )
    def pool(ctx_hbm, tab_hbm, st_hbm, en_hbm, out_hbm,
             st_v, en_v, rows_v, acc_v, sem_a, sem_b):
        wid = lax.axis_index("s") * 2 + lax.axis_index("c")
        base = wid * SPW
        b = base // NS
        r0 = base % NS
        pltpu.sync_copy(st_hbm.at[pl.ds(base, SPW)], st_v)
        pltpu.sync_copy(en_hbm.at[pl.ds(base, SPW)], en_v)
        lanes = lax.iota(jnp.int32, L)
        neg_vec = jnp.full((L,), NEG, jnp.float32)
        zero_vec = jnp.zeros((L,), jnp.float32)

        def get(vref, j):
            v = jnp.where(j >= L, vref[pl.ds(L, L)], vref[pl.ds(0, L)])
            return jnp.max(jnp.where(lanes == j % L, v, 0))

        def fire_span(j, slot, sem):
            s = get(st_v, j)
            e = get(en_v, j)
            s8 = (s + K - 1) // K
            e8 = e // K
            nb = e8 - s8
            has_interior = nb > 0
            lvl = lax.while_loop(lambda l_: (2 << l_) <= nb,
                                 lambda l_: l_ + 1, 0)
            w = 1 << lvl
            ni = jnp.where(has_interior, 2, 0)
            n1 = jnp.minimum(e, s8 * K) - s
            lo2 = jnp.maximum(s, e8 * K)
            n2 = e - lo2

            @pl.when(has_interior)
            def _():
                pltpu.async_copy(tab_hbm.at[b, lvl * NB + s8],
                                 rows_v.at[slot, 0], sem)
                pltpu.async_copy(tab_hbm.at[b, lvl * NB + e8 - w],
                                 rows_v.at[slot, 1], sem)

            def fire1(i, c):
                pltpu.async_copy(ctx_hbm.at[b, s + i],
                                 rows_v.at[slot, ni + i], sem)
                return c
            lax.fori_loop(0, n1, fire1, 0)

            def fire2(i, c):
                pltpu.async_copy(ctx_hbm.at[b, lo2 + i],
                                 rows_v.at[slot, ni + n1 + i], sem)
                return c
            lax.fori_loop(0, n2, fire2, 0)
            return ((ni + n1 + n2).astype(jnp.int32),
                    (e > s).astype(jnp.int32))

        def finish_span(j, slot, sem, meta):
            ntot, nonempty = meta

            def drain(i, c):
                pltpu.make_async_copy(ctx_hbm.at[b, 0],
                                      rows_v.at[0, 0], sem).wait()
                return c
            lax.fori_loop(0, ntot, drain, 0)

            for half in range(2):
                def acc_row(i, regs):
                    return tuple(
                        jnp.maximum(regs[m],
                                    rows_v[slot, i,
                                           pl.ds((half * (NV // 2) + m) * L, L)])
                        for m in range(NV // 2))
                regs = lax.fori_loop(0, ntot, acc_row,
                                     tuple(neg_vec for _ in range(NV // 2)))
                for m in range(NV // 2):
                    acc_v[pl.ds((half * (NV // 2) + m) * L, L)] = jnp.where(
                        nonempty > 0, regs[m], zero_vec)
            pltpu.sync_copy(acc_v, out_hbm.at[b, r0 + j])

        meta0 = fire_span(0, 0, sem_a)

        def pair_body(jj, meta_a):
            ja = 2 * jj
            meta_b = fire_span(ja + 1, 1, sem_b)
            finish_span(ja, 0, sem_a, meta_a)
            meta_next = lax.cond(
                ja + 2 < SPW,
                lambda: fire_span(ja + 2, 0, sem_a),
                lambda: (jnp.zeros((), jnp.int32), jnp.zeros((), jnp.int32)))
            finish_span(ja + 1, 1, sem_b, meta_b)
            return meta_next

        lax.fori_loop(0, SPW // 2, pair_body, meta0)

    return pool(context, table, starts, ends)


def kernel(context, spans):
    spans = spans.astype(jnp.int32)
    starts = spans[:, :, 0].reshape(B * NS)
    ends = spans[:, :, 1].reshape(B * NS)
    table = _build_table(context)
    pooled = _sc_pool(context, table, starts, ends)
    return jnp.zeros((B, S, D), jnp.float32).at[:, :NS, :].set(pooled)


# speedup vs baseline: 36.2476x; 1.0710x over previous
import dataclasses
import functools

import jax
import jax.numpy as jnp
from jax import lax
from jax.experimental import pallas as pl
from jax.experimental.pallas import tpu as pltpu
from jax.experimental.pallas import tpu_sc as plsc

B, S, D = 4, 4096, 1024
NS = 256
K = 8
NB = S // K
NSCALE = 9
TROWS = 1024
OFF = [TROWS - (TROWS >> j) for j in range(NSCALE)]
L = 16
NV = D // L
MAXROWS = 32
NW = 32
SPW = (B * NS) // NW
NEG = float(jnp.finfo(jnp.float32).min)
DH = 256


def _table_body(x_ref, t_ref, z_ref):
    x = x_ref[0]
    cur = jnp.max(x.reshape(NB, K, DH), axis=1)
    t_ref[0, 0:NB] = cur
    for j in range(1, NSCALE):
        n = NB >> j
        cur = jnp.max(cur.reshape(n, 2, DH), axis=1)
        t_ref[0, OFF[j]:OFF[j] + n] = cur
    t_ref[0, TROWS - 2:TROWS] = cur
    z_ref[0] = jnp.zeros((S, DH), jnp.float32)


def _build_table(context):
    return pl.pallas_call(
        _table_body,
        grid=(B, D // DH),
        in_specs=[pl.BlockSpec((1, S, DH), lambda b, d: (b, 0, d))],
        out_specs=[pl.BlockSpec((1, TROWS, DH), lambda b, d: (b, 0, d)),
                   pl.BlockSpec((1, S, DH), lambda b, d: (b, 0, d))],
        out_shape=[jax.ShapeDtypeStruct((B, TROWS, D), jnp.float32),
                   jax.ShapeDtypeStruct((B, S, D), jnp.float32)],
    )(context)


def _sc_pool(context, table, starts, ends):
    mesh = plsc.VectorSubcoreMesh(core_axis_name="c", subcore_axis_name="s")
    cp = pltpu.CompilerParams()
    if "needs_layout_passes" in pltpu.CompilerParams.__dataclass_fields__:
        cp = dataclasses.replace(cp, needs_layout_passes=False)

    @functools.partial(
        pl.kernel,
        out_type=jax.ShapeDtypeStruct((B, NS, D), jnp.float32),
        mesh=mesh,
        compiler_params=cp,
        scratch_types=[
            pltpu.VMEM((SPW,), jnp.int32),
            pltpu.VMEM((SPW,), jnp.int32),
            pltpu.VMEM((2, MAXROWS, D), jnp.float32),
            pltpu.VMEM((D,), jnp.float32),
            pltpu.SemaphoreType.DMA,
            pltpu.SemaphoreType.DMA,
        ],
    )
    def pool(ctx_hbm, tab_hbm, st_hbm, en_hbm, out_hbm,
             st_v, en_v, rows_v, acc_v, sem_a, sem_b):
        wid = lax.axis_index("s") * 2 + lax.axis_index("c")
        base = wid * SPW
        b = base // NS
        r0 = base % NS
        pltpu.sync_copy(st_hbm.at[pl.ds(base, SPW)], st_v)
        pltpu.sync_copy(en_hbm.at[pl.ds(base, SPW)], en_v)
        lanes = lax.iota(jnp.int32, L)
        neg_vec = jnp.full((L,), NEG, jnp.float32)
        zero_vec = jnp.zeros((L,), jnp.float32)

        def get(vref, j):
            v = jnp.where(j >= L, vref[pl.ds(L, L)], vref[pl.ds(0, L)])
            return jnp.max(jnp.where(lanes == j % L, v, 0))

        def fire_span(j, slot, sem):
            s = get(st_v, j)
            e = get(en_v, j)
            a = (s + K - 1) // K
            bb = e // K
            c = jnp.zeros((), jnp.int32)

            def fire_tab(row, cond, c):
                @pl.when(cond)
                def _():
                    pltpu.async_copy(tab_hbm.at[b, row],
                                     rows_v.at[slot, c], sem)
                return jnp.where(cond, c + 1, c)

            for jj in range(NSCALE - 1):
                w = 1 << jj
                up = ((a & w) != 0) & (a < bb)
                c = fire_tab(OFF[jj] + (a >> jj), up, c)
                a = jnp.where(up, a + w, a)
                dn = ((bb & w) != 0) & (a < bb)
                c = fire_tab(OFF[jj] + ((bb - w) >> jj), dn, c)
                bb = jnp.where(dn, bb - w, bb)
            wtop = 1 << (NSCALE - 1)
            c = fire_tab(OFF[NSCALE - 1] + (a >> (NSCALE - 1)), a < bb, c)
            c = fire_tab(OFF[NSCALE - 1] + (a >> (NSCALE - 1)) + 1,
                         a + wtop < bb, c)

            s8 = (s + K - 1) // K
            e8 = e // K
            n1 = jnp.minimum(e, s8 * K) - s
            lo2 = jnp.maximum(s, e8 * K)
            n2 = e - lo2

            def fire1(i, cc):
                pltpu.async_copy(ctx_hbm.at[b, s + i],
                                 rows_v.at[slot, c + i], sem)
                return cc
            lax.fori_loop(0, n1, fire1, 0)

            def fire2(i, cc):
                pltpu.async_copy(ctx_hbm.at[b, lo2 + i],
                                 rows_v.at[slot, c + n1 + i], sem)
                return cc
            lax.fori_loop(0, n2, fire2, 0)
            return ((c + n1 + n2).astype(jnp.int32),
                    (e > s).astype(jnp.int32))

        def finish_span(j, slot, sem, meta):
            ntot, nonempty = meta

            def drain(i, cc):
                pltpu.make_async_copy(ctx_hbm.at[b, 0],
                                      rows_v.at[0, 0], sem).wait()
                return cc
            lax.fori_loop(0, ntot, drain, 0)

            for half in range(2):
                def acc_row(i, regs):
                    return tuple(
                        jnp.maximum(regs[m],
                                    rows_v[slot, i,
                                           pl.ds((half * (NV // 2) + m) * L, L)])
                        for m in range(NV // 2))
                regs = lax.fori_loop(0, ntot, acc_row,
                                     tuple(neg_vec for _ in range(NV // 2)))
                for m in range(NV // 2):
                    acc_v[pl.ds((half * (NV // 2) + m) * L, L)] = jnp.where(
                        nonempty > 0, regs[m], zero_vec)
            pltpu.sync_copy(acc_v, out_hbm.at[b, r0 + j])

        meta0 = fire_span(0, 0, sem_a)

        def pair_body(jj, meta_a):
            ja = 2 * jj
            meta_b = fire_span(ja + 1, 1, sem_b)
            finish_span(ja, 0, sem_a, meta_a)
            meta_next = lax.cond(
                ja + 2 < SPW,
                lambda: fire_span(ja + 2, 0, sem_a),
                lambda: (jnp.zeros((), jnp.int32), jnp.zeros((), jnp.int32)))
            finish_span(ja + 1, 1, sem_b, meta_b)
            return meta_next

        lax.fori_loop(0, SPW // 2, pair_body, meta0)

    return pool(context, table, starts, ends)


def kernel(context, spans):
    spans = spans.astype(jnp.int32)
    starts = spans[:, :, 0].reshape(B * NS)
    ends = spans[:, :, 1].reshape(B * NS)
    table, zeros_out = _build_table(context)
    pooled = _sc_pool(context, table, starts, ends)
    return lax.dynamic_update_slice(zeros_out, pooled, (0, 0, 0))


# speedup vs baseline: 44.4335x; 1.2258x over previous
import dataclasses
import functools

import jax
import jax.numpy as jnp
from jax import lax
from jax.experimental import pallas as pl
from jax.experimental.pallas import tpu as pltpu
from jax.experimental.pallas import tpu_sc as plsc

B, S, D = 4, 4096, 1024
NS = 256
K = 8
NB = S // K
NSCALE = 9
PROWS = 1024
OFF = [PROWS - (PROWS >> j) for j in range(NSCALE)]
N16 = NB // 16
NL16 = 5
TROWS = PROWS + NL16 * N16
L = 16
NV = D // L
MAXROWS = 24
NW = 32
SPW = (B * NS) // NW
NEG = float(jnp.finfo(jnp.float32).min)
DH = 256


def _table_body(x_ref, t_ref):
    x = x_ref[0]
    cur = jnp.max(x.reshape(NB, K, DH), axis=1)
    t_ref[0, 0:NB] = cur
    lat = None
    for j in range(1, NSCALE):
        n = NB >> j
        cur = jnp.max(cur.reshape(n, 2, DH), axis=1)
        t_ref[0, OFF[j]:OFF[j] + n] = cur
        if j == 4:
            lat = cur
    t_ref[0, PROWS - 2:PROWS] = cur
    for l in range(1, NL16 + 1):
        h = 1 << (l - 1)
        lat = jnp.maximum(lat, jnp.concatenate([lat[h:], lat[:h]], axis=0))
        o = PROWS + (l - 1) * N16
        t_ref[0, o:o + N16] = lat


def _build_table(context):
    return pl.pallas_call(
        _table_body,
        grid=(B, D // DH),
        in_specs=[pl.BlockSpec((1, S, DH), lambda b, d: (b, 0, d))],
        out_specs=pl.BlockSpec((1, TROWS, DH), lambda b, d: (b, 0, d)),
        out_shape=jax.ShapeDtypeStruct((B, TROWS, D), jnp.float32),
    )(context)


def _sc_pool(context, table, starts, ends):
    mesh = plsc.VectorSubcoreMesh(core_axis_name="c", subcore_axis_name="s")
    cp = pltpu.CompilerParams()
    if "needs_layout_passes" in pltpu.CompilerParams.__dataclass_fields__:
        cp = dataclasses.replace(cp, needs_layout_passes=False)

    @functools.partial(
        pl.kernel,
        out_type=jax.ShapeDtypeStruct((B, NS, D), jnp.float32),
        mesh=mesh,
        compiler_params=cp,
        scratch_types=[
            pltpu.VMEM((SPW,), jnp.int32),
            pltpu.VMEM((SPW,), jnp.int32),
            pltpu.VMEM((2, MAXROWS, D), jnp.float32),
            pltpu.VMEM((D,), jnp.float32),
            pltpu.SemaphoreType.DMA,
            pltpu.SemaphoreType.DMA,
        ],
    )
    def pool(ctx_hbm, tab_hbm, st_hbm, en_hbm, out_hbm,
             st_v, en_v, rows_v, acc_v, sem_a, sem_b):
        wid = lax.axis_index("s") * 2 + lax.axis_index("c")
        base = wid * SPW
        b = base // NS
        r0 = base % NS
        pltpu.sync_copy(st_hbm.at[pl.ds(base, SPW)], st_v)
        pltpu.sync_copy(en_hbm.at[pl.ds(base, SPW)], en_v)
        lanes = lax.iota(jnp.int32, L)
        neg_vec = jnp.full((L,), NEG, jnp.float32)
        zero_vec = jnp.zeros((L,), jnp.float32)

        def get(vref, j):
            v = jnp.where(j >= L, vref[pl.ds(L, L)], vref[pl.ds(0, L)])
            return jnp.max(jnp.where(lanes == j % L, v, 0))

        def fire_span(j, slot, sem):
            s = get(st_v, j)
            e = get(en_v, j)
            a = (s + K - 1) // K
            bb = e // K
            c = jnp.zeros((), jnp.int32)

            def fire_tab(row, cond, c):
                @pl.when(cond)
                def _():
                    pltpu.async_copy(tab_hbm.at[b, row],
                                     rows_v.at[slot, c], sem)
                return jnp.where(cond, c + 1, c)

            for jj in range(4):
                w = 1 << jj
                up = ((a & w) != 0) & (a < bb)
                c = fire_tab(OFF[jj] + (a >> jj), up, c)
                a = jnp.where(up, a + w, a)
                dn = ((bb & w) != 0) & (a < bb)
                c = fire_tab(OFF[jj] + ((bb - w) >> jj), dn, c)
                bb = jnp.where(dn, bb - w, bb)
            nb16 = (bb - a) >> 4
            l16 = lax.while_loop(lambda l_: (2 << l_) <= nb16,
                                 lambda l_: l_ + 1, 0)
            w16 = 1 << l16
            i1 = a >> 4
            i2 = (bb >> 4) - w16
            row1 = jnp.where(l16 == 0, OFF[4] + i1,
                             PROWS + (l16 - 1) * N16 + i1)
            row2 = jnp.where(l16 == 0, OFF[4] + i2,
                             PROWS + (l16 - 1) * N16 + i2)
            c = fire_tab(row1, nb16 > 0, c)
            c = fire_tab(row2, nb16 > 0, c)

            s8 = (s + K - 1) // K
            e8 = e // K
            n1 = jnp.minimum(e, s8 * K) - s
            lo2 = jnp.maximum(s, e8 * K)
            n2 = e - lo2

            def fire1(i, cc):
                pltpu.async_copy(ctx_hbm.at[b, s + i],
                                 rows_v.at[slot, c + i], sem)
                return cc
            lax.fori_loop(0, n1, fire1, 0)

            def fire2(i, cc):
                pltpu.async_copy(ctx_hbm.at[b, lo2 + i],
                                 rows_v.at[slot, c + n1 + i], sem)
                return cc
            lax.fori_loop(0, n2, fire2, 0)
            return ((c + n1 + n2).astype(jnp.int32),
                    (e > s).astype(jnp.int32))

        def finish_span(j, slot, sem, meta):
            ntot, nonempty = meta

            def drain(i, cc):
                pltpu.make_async_copy(ctx_hbm.at[b, 0],
                                      rows_v.at[0, 0], sem).wait()
                return cc
            lax.fori_loop(0, ntot, drain, 0)

            for half in range(2):
                def acc_row(i, regs):
                    return tuple(
                        jnp.maximum(regs[m],
                                    rows_v[slot, i,
                                           pl.ds((half * (NV // 2) + m) * L, L)])
                        for m in range(NV // 2))
                regs = lax.fori_loop(0, ntot, acc_row,
                                     tuple(neg_vec for _ in range(NV // 2)))
                for m in range(NV // 2):
                    acc_v[pl.ds((half * (NV // 2) + m) * L, L)] = jnp.where(
                        nonempty > 0, regs[m], zero_vec)
            pltpu.sync_copy(acc_v, out_hbm.at[b, r0 + j])

        meta0 = fire_span(0, 0, sem_a)

        def pair_body(jj, meta_a):
            ja = 2 * jj
            meta_b = fire_span(ja + 1, 1, sem_b)
            finish_span(ja, 0, sem_a, meta_a)
            meta_next = lax.cond(
                ja + 2 < SPW,
                lambda: fire_span(ja + 2, 0, sem_a),
                lambda: (jnp.zeros((), jnp.int32), jnp.zeros((), jnp.int32)))
            finish_span(ja + 1, 1, sem_b, meta_b)
            return meta_next

        lax.fori_loop(0, SPW // 2, pair_body, meta0)

    return pool(context, table, starts, ends)


def kernel(context, spans):
    spans = spans.astype(jnp.int32)
    starts = spans[:, :, 0].reshape(B * NS)
    ends = spans[:, :, 1].reshape(B * NS)
    table = _build_table(context)
    pooled = _sc_pool(context, table, starts, ends)
    zeros_out = jnp.zeros((B, S, D), jnp.float32)
    return lax.dynamic_update_slice(zeros_out, pooled, (0, 0, 0))


# speedup vs baseline: 44.5288x; 1.0021x over previous
import dataclasses
import functools

import jax
import jax.numpy as jnp
from jax import lax
from jax.experimental import pallas as pl
from jax.experimental.pallas import tpu as pltpu
from jax.experimental.pallas import tpu_sc as plsc

B, S, D = 4, 4096, 1024
NS = 256
K = 8
NB = S // K
NSCALE = 9
PROWS = 1024
OFF = [PROWS - (PROWS >> j) for j in range(NSCALE)]
N16 = NB // 16
NL16 = 5
TROWS = PROWS + NL16 * N16
L = 16
NV = D // L
MAXROWS = 24
NW = 32
SPW = (B * NS) // NW
NEG = float(jnp.finfo(jnp.float32).min)
DH = 512


def _table_body(x_ref, t_ref):
    x = x_ref[0]
    cur = jnp.max(x.reshape(NB, K, DH), axis=1)
    t_ref[0, 0:NB] = cur
    lat = None
    for j in range(1, NSCALE):
        n = NB >> j
        cur = jnp.max(cur.reshape(n, 2, DH), axis=1)
        t_ref[0, OFF[j]:OFF[j] + n] = cur
        if j == 4:
            lat = cur
    t_ref[0, PROWS - 2:PROWS] = cur
    for l in range(1, NL16 + 1):
        h = 1 << (l - 1)
        lat = jnp.maximum(lat, jnp.concatenate([lat[h:], lat[:h]], axis=0))
        o = PROWS + (l - 1) * N16
        t_ref[0, o:o + N16] = lat


def _build_table(context):
    return pl.pallas_call(
        _table_body,
        grid=(B, D // DH),
        in_specs=[pl.BlockSpec((1, S, DH), lambda b, d: (b, 0, d))],
        out_specs=pl.BlockSpec((1, TROWS, DH), lambda b, d: (b, 0, d)),
        out_shape=jax.ShapeDtypeStruct((B, TROWS, D), jnp.float32),
    )(context)


def _sc_pool(context, table, starts, ends):
    mesh = plsc.VectorSubcoreMesh(core_axis_name="c", subcore_axis_name="s")
    cp = pltpu.CompilerParams()
    if "needs_layout_passes" in pltpu.CompilerParams.__dataclass_fields__:
        cp = dataclasses.replace(cp, needs_layout_passes=False)

    @functools.partial(
        pl.kernel,
        out_type=jax.ShapeDtypeStruct((B, NS, D), jnp.float32),
        mesh=mesh,
        compiler_params=cp,
        scratch_types=[
            pltpu.VMEM((SPW,), jnp.int32),
            pltpu.VMEM((SPW,), jnp.int32),
            pltpu.VMEM((2, MAXROWS, D), jnp.float32),
            pltpu.VMEM((D,), jnp.float32),
            pltpu.SemaphoreType.DMA,
            pltpu.SemaphoreType.DMA,
        ],
    )
    def pool(ctx_hbm, tab_hbm, st_hbm, en_hbm, out_hbm,
             st_v, en_v, rows_v, acc_v, sem_a, sem_b):
        wid = lax.axis_index("s") * 2 + lax.axis_index("c")
        base = wid * SPW
        b = base // NS
        r0 = base % NS
        pltpu.sync_copy(st_hbm.at[pl.ds(base, SPW)], st_v)
        pltpu.sync_copy(en_hbm.at[pl.ds(base, SPW)], en_v)
        lanes = lax.iota(jnp.int32, L)
        neg_vec = jnp.full((L,), NEG, jnp.float32)
        zero_vec = jnp.zeros((L,), jnp.float32)

        def get(vref, j):
            v = jnp.where(j >= L, vref[pl.ds(L, L)], vref[pl.ds(0, L)])
            return jnp.max(jnp.where(lanes == j % L, v, 0))

        def fire_span(j, slot, sem):
            s = get(st_v, j)
            e = get(en_v, j)
            a = (s + K - 1) // K
            bb = e // K
            c = jnp.zeros((), jnp.int32)

            def fire_tab(row, cond, c):
                @pl.when(cond)
                def _():
                    pltpu.async_copy(tab_hbm.at[b, row],
                                     rows_v.at[slot, c], sem)
                return jnp.where(cond, c + 1, c)

            for jj in range(4):
                w = 1 << jj
                up = ((a & w) != 0) & (a < bb)
                c = fire_tab(OFF[jj] + (a >> jj), up, c)
                a = jnp.where(up, a + w, a)
                dn = ((bb & w) != 0) & (a < bb)
                c = fire_tab(OFF[jj] + ((bb - w) >> jj), dn, c)
                bb = jnp.where(dn, bb - w, bb)
            nb16 = (bb - a) >> 4
            l16 = lax.while_loop(lambda l_: (2 << l_) <= nb16,
                                 lambda l_: l_ + 1, 0)
            w16 = 1 << l16
            i1 = a >> 4
            i2 = (bb >> 4) - w16
            row1 = jnp.where(l16 == 0, OFF[4] + i1,
                             PROWS + (l16 - 1) * N16 + i1)
            row2 = jnp.where(l16 == 0, OFF[4] + i2,
                             PROWS + (l16 - 1) * N16 + i2)
            c = fire_tab(row1, nb16 > 0, c)
            c = fire_tab(row2, nb16 > 0, c)

            s8 = (s + K - 1) // K
            e8 = e // K
            n1 = jnp.minimum(e, s8 * K) - s
            lo2 = jnp.maximum(s, e8 * K)
            n2 = e - lo2

            def fire1(i, cc):
                pltpu.async_copy(ctx_hbm.at[b, s + i],
                                 rows_v.at[slot, c + i], sem)
                return cc
            lax.fori_loop(0, n1, fire1, 0)

            def fire2(i, cc):
                pltpu.async_copy(ctx_hbm.at[b, lo2 + i],
                                 rows_v.at[slot, c + n1 + i], sem)
                return cc
            lax.fori_loop(0, n2, fire2, 0)
            return ((c + n1 + n2).astype(jnp.int32),
                    (e > s).astype(jnp.int32))

        def finish_span(j, slot, sem, meta):
            ntot, nonempty = meta

            def drain(i, cc):
                pltpu.make_async_copy(ctx_hbm.at[b, 0],
                                      rows_v.at[0, 0], sem).wait()
                return cc
            lax.fori_loop(0, ntot, drain, 0)

            for half in range(2):
                def acc_row(i, regs):
                    return tuple(
                        jnp.maximum(regs[m],
                                    rows_v[slot, i,
                                           pl.ds((half * (NV // 2) + m) * L, L)])
                        for m in range(NV // 2))
                regs = lax.fori_loop(0, ntot, acc_row,
                                     tuple(neg_vec for _ in range(NV // 2)))
                for m in range(NV // 2):
                    acc_v[pl.ds((half * (NV // 2) + m) * L, L)] = jnp.where(
                        nonempty > 0, regs[m], zero_vec)
            pltpu.sync_copy(acc_v, out_hbm.at[b, r0 + j])

        meta0 = fire_span(0, 0, sem_a)

        def pair_body(jj, meta_a):
            ja = 2 * jj
            meta_b = fire_span(ja + 1, 1, sem_b)
            finish_span(ja, 0, sem_a, meta_a)
            meta_next = lax.cond(
                ja + 2 < SPW,
                lambda: fire_span(ja + 2, 0, sem_a),
                lambda: (jnp.zeros((), jnp.int32), jnp.zeros((), jnp.int32)))
            finish_span(ja + 1, 1, sem_b, meta_b)
            return meta_next

        lax.fori_loop(0, SPW // 2, pair_body, meta0)

    return pool(context, table, starts, ends)


def kernel(context, spans):
    spans = spans.astype(jnp.int32)
    starts = spans[:, :, 0].reshape(B * NS)
    ends = spans[:, :, 1].reshape(B * NS)
    table = _build_table(context)
    zeros_out = lax.optimization_barrier(jnp.zeros((B, S, D), jnp.float32))
    pooled = _sc_pool(context, table, starts, ends)
    return lax.dynamic_update_slice(zeros_out, pooled, (0, 0, 0))


# speedup vs baseline: 47.5158x; 1.0671x over previous
import dataclasses
import functools

import jax
import jax.numpy as jnp
from jax import lax
from jax.experimental import pallas as pl
from jax.experimental.pallas import tpu as pltpu
from jax.experimental.pallas import tpu_sc as plsc

B, S, D = 4, 4096, 1024
NS = 256
K = 8
NB = S // K
NSCALE = 9
PROWS = 1024
OFF = [PROWS - (PROWS >> j) for j in range(NSCALE)]
N16 = NB // 16
NL16 = 5
TROWS = PROWS + NL16 * N16
L = 16
NV = D // L
MAXROWS = 24
NW = 32
SPW = (B * NS) // NW
NEG = float(jnp.finfo(jnp.float32).min)
DH = 512


def _table_body(x_ref, t_ref, z_ref):
    z_ref[0] = jnp.zeros((S, DH), jnp.float32)
    x = x_ref[0]
    cur = jnp.max(x.reshape(NB, K, DH), axis=1)
    t_ref[0, 0:NB] = cur
    lat = None
    for j in range(1, NSCALE):
        n = NB >> j
        cur = jnp.max(cur.reshape(n, 2, DH), axis=1)
        t_ref[0, OFF[j]:OFF[j] + n] = cur
        if j == 4:
            lat = cur
    t_ref[0, PROWS - 2:PROWS] = cur
    for l in range(1, NL16 + 1):
        h = 1 << (l - 1)
        lat = jnp.maximum(lat, jnp.concatenate([lat[h:], lat[:h]], axis=0))
        o = PROWS + (l - 1) * N16
        t_ref[0, o:o + N16] = lat


def _build_table(context):
    return pl.pallas_call(
        _table_body,
        grid=(B, D // DH),
        in_specs=[pl.BlockSpec((1, S, DH), lambda b, d: (b, 0, d))],
        out_specs=[pl.BlockSpec((1, TROWS, DH), lambda b, d: (b, 0, d)),
                   pl.BlockSpec((1, S, DH), lambda b, d: (b, 0, d))],
        out_shape=[jax.ShapeDtypeStruct((B, TROWS, D), jnp.float32),
                   jax.ShapeDtypeStruct((B, S, D), jnp.float32)],
    )(context)


def _sc_pool(context, table, starts, ends):
    mesh = plsc.VectorSubcoreMesh(core_axis_name="c", subcore_axis_name="s")
    cp = pltpu.CompilerParams()
    if "needs_layout_passes" in pltpu.CompilerParams.__dataclass_fields__:
        cp = dataclasses.replace(cp, needs_layout_passes=False)

    @functools.partial(
        pl.kernel,
        out_type=jax.ShapeDtypeStruct((B, NS, D), jnp.float32),
        mesh=mesh,
        compiler_params=cp,
        scratch_types=[
            pltpu.VMEM((SPW,), jnp.int32),
            pltpu.VMEM((SPW,), jnp.int32),
            pltpu.VMEM((2, MAXROWS, D), jnp.float32),
            pltpu.VMEM((D,), jnp.float32),
            pltpu.SemaphoreType.DMA,
            pltpu.SemaphoreType.DMA,
        ],
    )
    def pool(ctx_hbm, tab_hbm, st_hbm, en_hbm, out_hbm,
             st_v, en_v, rows_v, acc_v, sem_a, sem_b):
        wid = lax.axis_index("s") * 2 + lax.axis_index("c")
        base = wid * SPW
        b = base // NS
        r0 = base % NS
        pltpu.sync_copy(st_hbm.at[pl.ds(base, SPW)], st_v)
        pltpu.sync_copy(en_hbm.at[pl.ds(base, SPW)], en_v)
        lanes = lax.iota(jnp.int32, L)
        neg_vec = jnp.full((L,), NEG, jnp.float32)
        zero_vec = jnp.zeros((L,), jnp.float32)

        def get(vref, j):
            v = jnp.where(j >= L, vref[pl.ds(L, L)], vref[pl.ds(0, L)])
            return jnp.max(jnp.where(lanes == j % L, v, 0))

        def fire_span(j, slot, sem):
            s = get(st_v, j)
            e = get(en_v, j)
            a = (s + K - 1) // K
            bb = e // K
            c = jnp.zeros((), jnp.int32)

            def fire_tab(row, cond, c):
                @pl.when(cond)
                def _():
                    pltpu.async_copy(tab_hbm.at[b, row],
                                     rows_v.at[slot, c], sem)
                return jnp.where(cond, c + 1, c)

            for jj in range(4):
                w = 1 << jj
                up = ((a & w) != 0) & (a < bb)
                c = fire_tab(OFF[jj] + (a >> jj), up, c)
                a = jnp.where(up, a + w, a)
                dn = ((bb & w) != 0) & (a < bb)
                c = fire_tab(OFF[jj] + ((bb - w) >> jj), dn, c)
                bb = jnp.where(dn, bb - w, bb)
            nb16 = (bb - a) >> 4
            l16 = lax.while_loop(lambda l_: (2 << l_) <= nb16,
                                 lambda l_: l_ + 1, 0)
            w16 = 1 << l16
            i1 = a >> 4
            i2 = (bb >> 4) - w16
            row1 = jnp.where(l16 == 0, OFF[4] + i1,
                             PROWS + (l16 - 1) * N16 + i1)
            row2 = jnp.where(l16 == 0, OFF[4] + i2,
                             PROWS + (l16 - 1) * N16 + i2)
            c = fire_tab(row1, nb16 > 0, c)
            c = fire_tab(row2, nb16 > 0, c)

            s8 = (s + K - 1) // K
            e8 = e // K
            n1 = jnp.minimum(e, s8 * K) - s
            lo2 = jnp.maximum(s, e8 * K)
            n2 = e - lo2

            def fire1(i, cc):
                pltpu.async_copy(ctx_hbm.at[b, s + i],
                                 rows_v.at[slot, c + i], sem)
                return cc
            lax.fori_loop(0, n1, fire1, 0)

            def fire2(i, cc):
                pltpu.async_copy(ctx_hbm.at[b, lo2 + i],
                                 rows_v.at[slot, c + n1 + i], sem)
                return cc
            lax.fori_loop(0, n2, fire2, 0)
            return ((c + n1 + n2).astype(jnp.int32),
                    (e > s).astype(jnp.int32))

        def finish_span(j, slot, sem, meta):
            ntot, nonempty = meta

            def drain(i, cc):
                pltpu.make_async_copy(ctx_hbm.at[b, 0],
                                      rows_v.at[0, 0], sem).wait()
                return cc
            lax.fori_loop(0, ntot, drain, 0)

            for half in range(2):
                def acc_row(i, regs):
                    return tuple(
                        jnp.maximum(regs[m],
                                    rows_v[slot, i,
                                           pl.ds((half * (NV // 2) + m) * L, L)])
                        for m in range(NV // 2))
                regs = lax.fori_loop(0, ntot, acc_row,
                                     tuple(neg_vec for _ in range(NV // 2)))
                for m in range(NV // 2):
                    acc_v[pl.ds((half * (NV // 2) + m) * L, L)] = jnp.where(
                        nonempty > 0, regs[m], zero_vec)
            pltpu.sync_copy(acc_v, out_hbm.at[b, r0 + j])

        meta0 = fire_span(0, 0, sem_a)

        def pair_body(jj, meta_a):
            ja = 2 * jj
            meta_b = fire_span(ja + 1, 1, sem_b)
            finish_span(ja, 0, sem_a, meta_a)
            meta_next = lax.cond(
                ja + 2 < SPW,
                lambda: fire_span(ja + 2, 0, sem_a),
                lambda: (jnp.zeros((), jnp.int32), jnp.zeros((), jnp.int32)))
            finish_span(ja + 1, 1, sem_b, meta_b)
            return meta_next

        lax.fori_loop(0, SPW // 2, pair_body, meta0)

    return pool(context, table, starts, ends)


def kernel(context, spans):
    spans = spans.astype(jnp.int32)
    starts = spans[:, :, 0].reshape(B * NS)
    ends = spans[:, :, 1].reshape(B * NS)
    table, zeros_out = _build_table(context)
    pooled = _sc_pool(context, table, starts, ends)
    return lax.dynamic_update_slice(zeros_out, pooled, (0, 0, 0))
